# baseline (speedup 1.0000x reference)
import functools

import jax
import jax.numpy as jnp
from jax import lax
from jax.experimental import pallas as pl
from jax.experimental.pallas import tpu as pltpu
from jax.experimental.pallas import tpu_sc as plsc

_L = 16
_CAP = 8192
_MIN32 = -2147483648


def _scan_hist2(hist, krem0, krem1):
    iota = lax.iota(jnp.int32, _L)
    state = []
    for krem in (krem0, krem1):
        state.append([jnp.int32(0), jnp.int32(256), jnp.int32(0), jnp.int32(0), krem])
    for vi in range(256 // _L):
        for c in (0, 1):
            carry, dstar, before, hsel, krem = state[c]
            hv = hist[pl.ds(c * 256 + vi * _L, _L)]
            g = carry + plsc.cumsum(hv)
            nb = jnp.sum((g < krem).astype(jnp.int32), axis=0)
            sel = iota == nb
            gd = jnp.sum(jnp.where(sel, g, 0), axis=0)
            hd = jnp.sum(jnp.where(sel, hv, 0), axis=0)
            first = jnp.logical_and(nb < _L, dstar == 256)
            state[c] = [
                carry + jnp.sum(hv, axis=0),
                jnp.where(first, vi * _L + nb, dstar),
                jnp.where(first, gd - hd, before),
                jnp.where(first, hd, hsel),
                krem,
            ]
    return [(s[1], s[2], s[3]) for s in state]


def _zero_hist(hist, zeros):
    for i in range(512 // _L):
        hist[pl.ds(i * _L, _L)] = zeros


def _refine_levels(get_kb, nv_of, p, krem, hist, zeros, ones, unroll):
    esel = [jnp.int32(0), jnp.int32(0)]
    for s_ in (16, 8, 0):
        _zero_hist(hist, zeros)

        @plsc.parallel_loop(0, nv_of, unroll=unroll)
        def _(i, _s=s_, _p=tuple(p)):
            for c in (0, 1):
                kb, valid = get_kb(c, i)
                pm = jnp.logical_and(
                    lax.shift_right_logical(kb, _s + 8) == _p[c], valid)
                d_ = (lax.shift_right_logical(kb, _s) & 255) + c * 256
                plsc.addupdate_scatter(hist, [d_], ones, mask=pm)

        res = _scan_hist2(hist, krem[0], krem[1])
        for c in (0, 1):
            dstar, nbefore, hsel = res[c]
            krem[c] = krem[c] - nbefore
            p[c] = lax.shift_left(p[c], 8) | dstar
            esel[c] = hsel
    return p, krem, esel


def _make_sc(n, d):
    cols = d // 32
    k = n // 2
    nv = n // _L
    mesh = plsc.VectorSubcoreMesh(core_axis_name="c", subcore_axis_name="s")

    @functools.partial(
        pl.kernel,
        mesh=mesh,
        out_type=jax.ShapeDtypeStruct((d, n), jnp.float32),
        compiler_params=pltpu.CompilerParams(needs_layout_passes=False),
        scratch_types=[
            pltpu.VMEM((cols, n), jnp.float32),
            pltpu.VMEM((cols, n), jnp.int32),
            pltpu.VMEM((cols, n), jnp.float32),
            pltpu.VMEM((cols * (_CAP + _L),), jnp.int32),
            pltpu.VMEM((512,), jnp.int32),
            pltpu.SMEM((8,), jnp.int32),
        ],
    )
    def run(x_hbm, out_hbm, xv, keyv, outv, cand, hist, res_s):
        wid = lax.axis_index("s") * 2 + lax.axis_index("c")
        base = wid * cols
        pltpu.sync_copy(x_hbm.at[pl.ds(base, cols)], xv)

        ones = jnp.ones((_L,), jnp.int32)
        zeros = jnp.zeros((_L,), jnp.int32)
        iota = lax.iota(jnp.int32, _L)

        _zero_hist(hist, zeros)

        @plsc.parallel_loop(0, nv, unroll=8)
        def _(i):
            for c in (0, 1):
                xvec = xv[c, pl.ds(i * _L, _L)]
                ib = lax.bitcast_convert_type(xvec, jnp.int32)
                asc = jnp.where(ib >= 0, ib, -(ib & jnp.int32(0x7FFFFFFF)))
                kb = (~asc) ^ _MIN32
                keyv[c, pl.ds(i * _L, _L)] = kb
                d_ = (lax.shift_right_logical(kb, 24) & 255) + c * 256
                plsc.addupdate_scatter(hist, [d_], ones)

        res0 = _scan_hist2(hist, jnp.int32(k), jnp.int32(k))
        d0 = [res0[c][0] for c in (0, 1)]
        krem = [jnp.int32(k) - res0[c][1] for c in (0, 1)]
        e0 = [res0[c][2] for c in (0, 1)]
        p0 = list(d0)

        both_small = jnp.logical_and(e0[0] <= _CAP, e0[1] <= _CAP)

        @pl.when(both_small)
        def _():
            def body(i, off):
                off = list(off)
                for c in (0, 1):
                    kb = keyv[c, pl.ds(i * _L, _L)]
                    pm = lax.shift_right_logical(kb, 24) == d0[c]
                    pos = plsc.cumsum(pm.astype(jnp.int32))
                    idx = jnp.maximum(off[c] + pos - 1, 0) + c * (_CAP + _L)
                    plsc.store_scatter(cand, [idx], kb, mask=pm)
                    off[c] = off[c] + jnp.sum(pm.astype(jnp.int32), axis=0)
                return tuple(off)
            cnt = lax.fori_loop(0, nv, body, (jnp.int32(0), jnp.int32(0)))

            for c in (0, 1):
                fill = jnp.where(d0[c] == 0, jnp.int32(1 << 24), jnp.int32(0))
                cand[pl.ds(c * (_CAP + _L) + cnt[c], _L)] = (
                    jnp.full((_L,), 0, jnp.int32) + fill)

            ncv = jnp.maximum((cnt[0] + _L - 1) // _L, (cnt[1] + _L - 1) // _L)

            def get_kb(c, i):
                kb = cand[pl.ds(c * (_CAP + _L) + i * _L, _L)]
                valid = (i * _L + iota) < cnt[c]
                return kb, valid

            p, kr, es = _refine_levels(
                get_kb, ncv, list(p0), list(krem), hist, zeros, ones, 1)
            for c in (0, 1):
                res_s[c] = p[c]
                res_s[2 + c] = kr[c]
                res_s[4 + c] = es[c]

        @pl.when(jnp.logical_not(both_small))
        def _():
            def get_kb(c, i):
                return keyv[c, pl.ds(i * _L, _L)], True

            p, kr, es = _refine_levels(
                get_kb, nv, list(p0), list(krem), hist, zeros, ones, 8)
            for c in (0, 1):
                res_s[c] = p[c]
                res_s[2 + c] = kr[c]
                res_s[4 + c] = es[c]

        ts = [res_s[0] ^ _MIN32, res_s[1] ^ _MIN32]
        m = [res_s[2], res_s[3]]
        esel = [res_s[4], res_s[5]]
        both_fast = jnp.logical_and(m[0] == esel[0], m[1] == esel[1])

        @pl.when(both_fast)
        def _():
            @plsc.parallel_loop(0, nv, unroll=8)
            def _(i):
                for c in (0, 1):
                    ks = keyv[c, pl.ds(i * _L, _L)] ^ _MIN32
                    outv[c, pl.ds(i * _L, _L)] = jnp.where(
                        ks <= ts[c], jnp.float32(1.0), jnp.float32(-1.0))

        @pl.when(jnp.logical_not(both_fast))
        def _():
            for c in (0, 1):
                def body(i, cnt2, _c=c):
                    ks = keyv[_c, pl.ds(i * _L, _L)] ^ _MIN32
                    eqm = ks == ts[_c]
                    eqi = eqm.astype(jnp.int32)
                    pos = cnt2 + plsc.cumsum(eqi)
                    take = (ks < ts[_c]) | (eqm & (pos <= m[_c]))
                    outv[_c, pl.ds(i * _L, _L)] = jnp.where(
                        take, jnp.float32(1.0), jnp.float32(-1.0))
                    return cnt2 + jnp.sum(eqi, axis=0)
                lax.fori_loop(0, nv, body, jnp.int32(0))

        pltpu.sync_copy(outv, out_hbm.at[pl.ds(base, cols)])

    return run


@jax.jit
def kernel(x):
    n, d = x.shape
    out_t = _make_sc(n, d)(x.T)
    return out_t.T


# speedup vs baseline: 1.4998x; 1.4998x over previous
import functools

import jax
import jax.numpy as jnp
from jax import lax
from jax.experimental import pallas as pl
from jax.experimental.pallas import tpu as pltpu
from jax.experimental.pallas import tpu_sc as plsc

_L = 16
_MIN32 = -2147483648


def _scan_hist2(hist, krem0, krem1):
    iota = lax.iota(jnp.int32, _L)
    state = []
    for krem in (krem0, krem1):
        state.append([jnp.int32(0), jnp.int32(256), jnp.int32(0), jnp.int32(0), krem])
    for vi in range(256 // _L):
        for c in (0, 1):
            carry, dstar, before, hsel, krem = state[c]
            hv = hist[pl.ds(c * 256 + vi * _L, _L)]
            g = carry + plsc.cumsum(hv)
            nb = jnp.sum((g < krem).astype(jnp.int32), axis=0)
            sel = iota == nb
            gd = jnp.sum(jnp.where(sel, g, 0), axis=0)
            hd = jnp.sum(jnp.where(sel, hv, 0), axis=0)
            first = jnp.logical_and(nb < _L, dstar == 256)
            state[c] = [
                carry + jnp.sum(hv, axis=0),
                jnp.where(first, vi * _L + nb, dstar),
                jnp.where(first, gd - hd, before),
                jnp.where(first, hd, hsel),
                krem,
            ]
    return [(s[1], s[2], s[3]) for s in state]


def _pair(n, xv, keyv, outv, hist):
    k = n // 2
    nv = n // _L
    ones = jnp.ones((_L,), jnp.int32)
    zeros = jnp.zeros((_L,), jnp.int32)

    p = [jnp.int32(0), jnp.int32(0)]
    krem = [jnp.int32(k), jnp.int32(k)]
    esel = [jnp.int32(0), jnp.int32(0)]

    for li, s in enumerate((24, 16, 8, 0)):
        for i in range(512 // _L):
            hist[pl.ds(i * _L, _L)] = zeros

        if li == 0:
            @plsc.parallel_loop(0, nv, unroll=8)
            def _(i):
                for c in (0, 1):
                    xvec = xv[c, pl.ds(i * _L, _L)]
                    ib = lax.bitcast_convert_type(xvec, jnp.int32)
                    asc = jnp.where(ib >= 0, ib, -(ib & jnp.int32(0x7FFFFFFF)))
                    kb = (~asc) ^ _MIN32
                    keyv[c, pl.ds(i * _L, _L)] = kb
                    d = (lax.shift_right_logical(kb, 24) & 255) + c * 256
                    plsc.addupdate_scatter(hist, [d], ones)
        else:
            @plsc.parallel_loop(0, nv, unroll=8)
            def _(i, _s=s, _p=tuple(p)):
                for c in (0, 1):
                    kb = keyv[c, pl.ds(i * _L, _L)]
                    pm = lax.shift_right_logical(kb, _s + 8) == _p[c]
                    d = (lax.shift_right_logical(kb, _s) & 255) + c * 256
                    plsc.addupdate_scatter(hist, [d], ones, mask=pm)

        res = _scan_hist2(hist, krem[0], krem[1])
        for c in (0, 1):
            dstar, nbefore, hsel = res[c]
            krem[c] = krem[c] - nbefore
            p[c] = lax.shift_left(p[c], 8) | dstar
            esel[c] = hsel

    ts = [p[0] ^ _MIN32, p[1] ^ _MIN32]
    m = krem
    both_fast = jnp.logical_and(m[0] == esel[0], m[1] == esel[1])

    @pl.when(both_fast)
    def _():
        @plsc.parallel_loop(0, nv, unroll=8)
        def _(i):
            for c in (0, 1):
                ks = keyv[c, pl.ds(i * _L, _L)] ^ _MIN32
                outv[c, pl.ds(i * _L, _L)] = jnp.where(
                    ks <= ts[c], jnp.float32(1.0), jnp.float32(-1.0))

    @pl.when(jnp.logical_not(both_fast))
    def _():
        for c in (0, 1):
            def body(i, cnt, _c=c):
                ks = keyv[_c, pl.ds(i * _L, _L)] ^ _MIN32
                eqm = ks == ts[_c]
                eqi = eqm.astype(jnp.int32)
                pos = cnt + plsc.cumsum(eqi)
                take = (ks < ts[_c]) | (eqm & (pos <= m[_c]))
                outv[_c, pl.ds(i * _L, _L)] = jnp.where(
                    take, jnp.float32(1.0), jnp.float32(-1.0))
                return cnt + jnp.sum(eqi, axis=0)
            lax.fori_loop(0, nv, body, jnp.int32(0))


def _make_sc(n, d):
    cols = d // 32
    mesh = plsc.VectorSubcoreMesh(core_axis_name="c", subcore_axis_name="s")

    @functools.partial(
        pl.kernel,
        mesh=mesh,
        out_type=jax.ShapeDtypeStruct((d, n), jnp.float32),
        compiler_params=pltpu.CompilerParams(needs_layout_passes=False),
        scratch_types=[
            pltpu.VMEM((cols, n), jnp.float32),
            pltpu.VMEM((cols, n), jnp.int32),
            pltpu.VMEM((cols, n), jnp.float32),
            pltpu.VMEM((512,), jnp.int32),
        ],
    )
    def run(x_hbm, out_hbm, xv, keyv, outv, hist):
        wid = lax.axis_index("s") * 2 + lax.axis_index("c")
        base = wid * cols
        pltpu.sync_copy(x_hbm.at[pl.ds(base, cols)], xv)
        _pair(n, xv, keyv, outv, hist)
        pltpu.sync_copy(outv, out_hbm.at[pl.ds(base, cols)])

    return run


@jax.jit
def kernel(x):
    n, d = x.shape
    out_t = _make_sc(n, d)(x.T)
    return out_t.T
